# baseline (speedup 1.0000x reference)
import functools

import jax
import jax.numpy as jnp
from jax import lax
from jax.experimental import pallas as pl
from jax.experimental.pallas import tpu as pltpu
from jax.experimental.pallas import tpu_sc as plsc

N = 500000
CHANNELS = 64
NUM_EMB = 8

BLK = 800
NBLK = N // BLK
SUB = 80
NSUB = BLK // SUB
LANES = 16

_info = plsc.get_sparse_core_info()
NC, NS = _info.num_cores, _info.num_subcores
NW = NC * NS


def _body(coords_hbm, table_hbm, out_hbm, coords_v, idx_v, rows_v, sem):
    wid = lax.axis_index("s") * NC + lax.axis_index("c")
    nb = (NBLK - 1 - wid) // NW + 1
    iota = jnp.arange(LANES, dtype=jnp.int32)

    def blk_body(t, carry):
        base = (wid + t * NW) * BLK
        pltpu.sync_copy(coords_hbm.at[pl.ds(base * 4, BLK * 4)], coords_v)
        for j in range(NSUB):
            for k in range(SUB // LANES):
                g = j * (SUB // LANES) + k
                rows = iota * 4 + (g * LANES * 4)
                c1 = plsc.load_gather(coords_v, [rows + 1])
                c2 = plsc.load_gather(coords_v, [rows + 2])
                c3 = plsc.load_gather(coords_v, [rows + 3])
                idx = (c1 & 1) | ((c2 & 1) << 1) | ((c3 & 1) << 2)
                idx_v[j, pl.ds(k * LANES, LANES)] = idx
        copies = [
            pltpu.async_copy(
                table_hbm.at[idx_v.at[j]], rows_v.at[pl.ds(j * SUB, SUB)], sem
            )
            for j in range(NSUB)
        ]
        for c in copies:
            c.wait()
        pltpu.sync_copy(rows_v, out_hbm.at[pl.ds(base, BLK)])
        return carry

    lax.fori_loop(0, nb, blk_body, 0)


@functools.partial(jax.jit, donate_argnums=())
def kernel(x_coords, emb_table):
    mesh = plsc.VectorSubcoreMesh(core_axis_name="c", subcore_axis_name="s")
    f = functools.partial(
        pl.kernel,
        out_type=jax.ShapeDtypeStruct((N, CHANNELS), jnp.float32),
        mesh=mesh,
        compiler_params=pltpu.CompilerParams(
            needs_layout_passes=False, use_tc_tiling_on_sc=False
        ),
        scratch_types=[
            pltpu.VMEM((BLK * 4,), jnp.int32),
            pltpu.VMEM((NSUB, SUB), jnp.int32),
            pltpu.VMEM((BLK, CHANNELS), jnp.float32),
            pltpu.SemaphoreType.DMA,
        ],
    )(_body)
    return f(x_coords.reshape(-1), emb_table)


# speedup vs baseline: 2.8372x; 2.8372x over previous
import functools

import jax
import jax.numpy as jnp
from jax import lax
from jax.experimental import pallas as pl
from jax.experimental.pallas import tpu as pltpu
from jax.experimental.pallas import tpu_sc as plsc

N = 500000
CHANNELS = 64
NUM_EMB = 8

BLK = 800
NBLK = N // BLK
GRP = BLK // 16
LANES = 16

_info = plsc.get_sparse_core_info()
NC, NS = _info.num_cores, _info.num_subcores
NW = NC * NS


def _body(coords_hbm, table_hbm, out_hbm, coords_v, table_v, rows_v, sem):
    wid = lax.axis_index("s") * NC + lax.axis_index("c")
    nb = (NBLK - 1 - wid) // NW + 1
    iota = jnp.arange(LANES, dtype=jnp.int32)
    c_idx = [iota * 4 + c for c in (1, 2, 3)]
    j_idx = [iota + j * LANES for j in range(4)]
    p_sel = [jnp.full((LANES,), p, jnp.int32) for p in range(LANES)]

    pltpu.sync_copy(table_hbm, table_v)
    cp0 = pltpu.async_copy(coords_hbm.at[pl.ds(wid * BLK * 4, BLK * 4)], coords_v, sem)

    def blk_body(t, carry):
        blk = wid + t * NW
        pltpu.make_async_copy(
            coords_hbm.at[pl.ds(blk * BLK * 4, BLK * 4)], coords_v, sem
        ).wait()

        def grp_body(g, carry2):
            base = g * (LANES * 4)
            c1 = plsc.load_gather(coords_v, [c_idx[0] + base])
            c2 = plsc.load_gather(coords_v, [c_idx[1] + base])
            c3 = plsc.load_gather(coords_v, [c_idx[2] + base])
            gbase = ((c1 & 1) << 6) | ((c2 & 1) << 7) | ((c3 & 1) << 8)
            goff = g * (LANES * CHANNELS)
            for p in range(LANES):
                bp = lax.gather(
                    gbase,
                    p_sel[p][:, None],
                    lax.GatherDimensionNumbers(
                        offset_dims=(), collapsed_slice_dims=(0,), start_index_map=(0,)
                    ),
                    (1,),
                    mode=lax.GatherScatterMode.PROMISE_IN_BOUNDS,
                )
                for j in range(4):
                    v = plsc.load_gather(table_v, [bp + j_idx[j]])
                    rows_v[pl.ds(goff + p * CHANNELS + j * LANES, LANES)] = v
            return carry2

        lax.fori_loop(0, GRP, grp_body, 0, unroll=2)

        @pl.when(t + 1 < nb)
        def _prefetch():
            pltpu.async_copy(
                coords_hbm.at[pl.ds((blk + NW) * BLK * 4, BLK * 4)], coords_v, sem
            )

        pltpu.sync_copy(rows_v, out_hbm.at[pl.ds(blk * BLK * CHANNELS, BLK * CHANNELS)])
        return carry

    lax.fori_loop(0, nb, blk_body, 0)
    del cp0


@functools.partial(jax.jit, donate_argnums=())
def kernel(x_coords, emb_table):
    mesh = plsc.VectorSubcoreMesh(core_axis_name="c", subcore_axis_name="s")
    f = functools.partial(
        pl.kernel,
        out_type=jax.ShapeDtypeStruct((N * CHANNELS,), jnp.float32),
        mesh=mesh,
        compiler_params=pltpu.CompilerParams(
            needs_layout_passes=False, use_tc_tiling_on_sc=False
        ),
        scratch_types=[
            pltpu.VMEM((BLK * 4,), jnp.int32),
            pltpu.VMEM((NUM_EMB * CHANNELS,), jnp.float32),
            pltpu.VMEM((BLK * CHANNELS,), jnp.float32),
            pltpu.SemaphoreType.DMA,
        ],
    )(_body)
    out = f(x_coords.reshape(-1), emb_table.reshape(-1))
    return out.reshape(N, CHANNELS)


# speedup vs baseline: 4.3813x; 1.5442x over previous
import functools

import jax
import jax.numpy as jnp
from jax import lax
from jax.experimental import pallas as pl
from jax.experimental.pallas import tpu as pltpu
from jax.experimental.pallas import tpu_sc as plsc

N = 500000
CHANNELS = 64
NUM_EMB = 8
LANES = 16

IR = 128
JT, JR = 8, 8
IT = (N + IR - 1) // IR
NPAD = IT * IR

TPB = 4
BLOCKS = (IT + TPB - 1) // TPB
GRPB = TPB * IR // LANES
CELEM = TPB * IR * 4
STRIPE = TPB * IR * JR

_info = plsc.get_sparse_core_info()
NC, NS = _info.num_cores, _info.num_subcores
NW = NC * NS


def _body(coords_hbm, table_hbm, out_hbm, coords_v, table_v, rows_v, sem_c, sem_o):
    wid = lax.axis_index("s") * NC + lax.axis_index("c")
    nb = (BLOCKS - 1 - wid) // NW + 1
    pltpu.sync_copy(table_hbm, table_v)

    def tile_start_of(b):
        return jnp.minimum(b * TPB, IT - TPB)

    def coords_issue(b, buf):
        pltpu.async_copy(
            coords_hbm.at[pl.ds(tile_start_of(b) * IR * 4, CELEM)],
            coords_v.at[pl.ds(buf * CELEM, CELEM)],
            sem_c,
        )

    def out_wait_one():
        pltpu.make_async_copy(
            rows_v.at[pl.ds(0, STRIPE)],
            out_hbm.at[pl.ds(0, STRIPE)],
            sem_o,
        ).wait()

    coords_issue(wid, 0)

    def blk_body(t, carry):
        b = wid + t * NW
        buf = t % 2
        ts = tile_start_of(b)

        @pl.when(t >= 2)
        def _drain():
            for _ in range(JT):
                out_wait_one()

        pltpu.make_async_copy(
            coords_hbm.at[pl.ds(ts * IR * 4, CELEM)],
            coords_v.at[pl.ds(buf * CELEM, CELEM)],
            sem_c,
        ).wait()

        @pl.when(t + 1 < nb)
        def _prefetch():
            coords_issue(b + NW, 1 - buf)

        cbase = buf * CELEM
        rbase = buf * (JT * STRIPE)

        def grp_body(g, carry2):
            cb = cbase + (g // 8) * (4 * IR) + (g % 8) * LANES
            c1 = coords_v[pl.ds(cb + 1 * IR, LANES)]
            c2 = coords_v[pl.ds(cb + 2 * IR, LANES)]
            c3 = coords_v[pl.ds(cb + 3 * IR, LANES)]
            gbase = ((c1 & 1) << 6) | ((c2 & 1) << 7) | ((c3 & 1) << 8)
            goff = rbase + (g // 8) * (JR * IR) + (g % 8) * LANES
            for j in range(CHANNELS):
                v = plsc.load_gather(table_v, [gbase + j])
                rows_v[pl.ds(goff + (j // JR) * STRIPE + (j % JR) * IR, LANES)] = v
            return carry2

        lax.fori_loop(0, GRPB, grp_body, 0, unroll=2)

        for jt in range(JT):
            pltpu.async_copy(
                rows_v.at[pl.ds(rbase + jt * STRIPE, STRIPE)],
                out_hbm.at[pl.ds(jt * (IT * JR * IR) + ts * (JR * IR), STRIPE)],
                sem_o,
            )
        return carry

    lax.fori_loop(0, nb, blk_body, 0)

    @pl.when(nb >= 1)
    def _drain_last():
        for _ in range(JT):
            out_wait_one()

    @pl.when(nb >= 2)
    def _drain_prev():
        for _ in range(JT):
            out_wait_one()


@functools.partial(jax.jit, donate_argnums=())
def kernel(x_coords, emb_table):
    mesh = plsc.VectorSubcoreMesh(core_axis_name="c", subcore_axis_name="s")
    f = functools.partial(
        pl.kernel,
        out_type=jax.ShapeDtypeStruct((JT * IT * JR * IR,), jnp.float32),
        mesh=mesh,
        compiler_params=pltpu.CompilerParams(
            needs_layout_passes=False, use_tc_tiling_on_sc=False
        ),
        scratch_types=[
            pltpu.VMEM((2 * CELEM,), jnp.int32),
            pltpu.VMEM((NUM_EMB * CHANNELS,), jnp.float32),
            pltpu.VMEM((2 * JT * STRIPE,), jnp.float32),
            pltpu.SemaphoreType.DMA,
            pltpu.SemaphoreType.DMA,
        ],
    )(_body)
    coords_pad = jnp.pad(x_coords, ((0, NPAD - N), (0, 0)))
    coords_flat = coords_pad.reshape(IT, IR, 4).transpose(0, 2, 1).reshape(-1)
    out_flat = f(coords_flat, emb_table.reshape(-1))
    p = out_flat.reshape(JT, IT, JR, IR)
    return p.transpose(1, 3, 0, 2).reshape(NPAD, CHANNELS)[:N]


# speedup vs baseline: 10.3201x; 2.3555x over previous
import functools

import jax
import jax.numpy as jnp
from jax import lax
from jax.experimental import pallas as pl
from jax.experimental.pallas import tpu as pltpu
from jax.experimental.pallas import tpu_sc as plsc

N = 500000
CHANNELS = 64
NUM_EMB = 8
LANES = 16

IR = 128
JT, JR = 8, 8
IT = (N + IR - 1) // IR
NPAD = IT * IR

TPB = 4
BLOCKS = (IT + TPB - 1) // TPB
GRPB = TPB * IR // LANES
CELEM = TPB * IR * 4
STRIPE = TPB * IR * JR

_info = plsc.get_sparse_core_info()
NC, NS = _info.num_cores, _info.num_subcores
NW = NC * NS

_GDIMS = lax.GatherDimensionNumbers(
    offset_dims=(), collapsed_slice_dims=(0,), start_index_map=(0,)
)


def _body(coords_hbm, table_hbm, out_hbm, coords_v, table_v, tt_v, rows_v, sem_c, sem_o):
    wid = lax.axis_index("s") * NC + lax.axis_index("c")
    nb = (BLOCKS - 1 - wid) // NW + 1
    iota = jnp.arange(LANES, dtype=jnp.int32)

    col_idx = (iota & 7) * CHANNELS

    def tt_body(j, carry):
        tt_v[pl.ds(j * LANES, LANES)] = plsc.load_gather(table_v, [col_idx + j])
        return carry

    def tile_start_of(b):
        return jnp.minimum(b * TPB, IT - TPB)

    def coords_issue(b, buf):
        pltpu.async_copy(
            coords_hbm.at[pl.ds(tile_start_of(b) * IR * 4, CELEM)],
            coords_v.at[pl.ds(buf * CELEM, CELEM)],
            sem_c,
        )

    def out_wait_one():
        pltpu.make_async_copy(
            rows_v.at[pl.ds(0, STRIPE)],
            out_hbm.at[pl.ds(0, STRIPE)],
            sem_o,
        ).wait()

    coords_issue(wid, 0)
    pltpu.sync_copy(table_hbm, table_v)
    lax.fori_loop(0, CHANNELS, tt_body, 0)

    def blk_body(t, carry):
        b = wid + t * NW
        buf = t % 2
        ts = tile_start_of(b)

        @pl.when(t >= 2)
        def _drain():
            for _ in range(JT):
                out_wait_one()

        pltpu.make_async_copy(
            coords_hbm.at[pl.ds(ts * IR * 4, CELEM)],
            coords_v.at[pl.ds(buf * CELEM, CELEM)],
            sem_c,
        ).wait()

        @pl.when(t + 1 < nb)
        def _prefetch():
            coords_issue(b + NW, 1 - buf)

        cbase = buf * CELEM
        rbase = buf * (JT * STRIPE)

        def grp_body(g, carry2):
            cb = cbase + (g // 8) * (4 * IR) + (g % 8) * LANES
            c1 = coords_v[pl.ds(cb + 1 * IR, LANES)]
            c2 = coords_v[pl.ds(cb + 2 * IR, LANES)]
            c3 = coords_v[pl.ds(cb + 3 * IR, LANES)]
            idx = (c1 & 1) | ((c2 & 1) << 1) | ((c3 & 1) << 2)
            sidx = idx[:, None]
            goff = rbase + (g // 8) * (JR * IR) + (g % 8) * LANES
            for j in range(CHANNELS):
                tcol = tt_v[pl.ds(j * LANES, LANES)]
                v = lax.gather(
                    tcol, sidx, _GDIMS, (1,),
                    mode=lax.GatherScatterMode.PROMISE_IN_BOUNDS,
                )
                rows_v[pl.ds(goff + (j // JR) * STRIPE + (j % JR) * IR, LANES)] = v
            return carry2

        lax.fori_loop(0, GRPB, grp_body, 0)

        for jt in range(JT):
            pltpu.async_copy(
                rows_v.at[pl.ds(rbase + jt * STRIPE, STRIPE)],
                out_hbm.at[pl.ds(jt * (IT * JR * IR) + ts * (JR * IR), STRIPE)],
                sem_o,
            )
        return carry

    lax.fori_loop(0, nb, blk_body, 0)

    @pl.when(nb >= 1)
    def _drain_last():
        for _ in range(JT):
            out_wait_one()

    @pl.when(nb >= 2)
    def _drain_prev():
        for _ in range(JT):
            out_wait_one()


@functools.partial(jax.jit, donate_argnums=())
def kernel(x_coords, emb_table):
    mesh = plsc.VectorSubcoreMesh(core_axis_name="c", subcore_axis_name="s")
    f = functools.partial(
        pl.kernel,
        out_type=jax.ShapeDtypeStruct((JT * IT * JR * IR,), jnp.float32),
        mesh=mesh,
        compiler_params=pltpu.CompilerParams(
            needs_layout_passes=False, use_tc_tiling_on_sc=False
        ),
        scratch_types=[
            pltpu.VMEM((2 * CELEM,), jnp.int32),
            pltpu.VMEM((NUM_EMB * CHANNELS,), jnp.float32),
            pltpu.VMEM((CHANNELS * LANES,), jnp.float32),
            pltpu.VMEM((2 * JT * STRIPE,), jnp.float32),
            pltpu.SemaphoreType.DMA,
            pltpu.SemaphoreType.DMA,
        ],
    )(_body)
    coords_pad = jnp.pad(x_coords, ((0, NPAD - N), (0, 0)))
    coords_flat = coords_pad.reshape(IT, IR, 4).transpose(0, 2, 1).reshape(-1)
    out_flat = f(coords_flat, emb_table.reshape(-1))
    p = out_flat.reshape(JT, IT, JR, IR)
    return p.transpose(1, 3, 0, 2).reshape(NPAD, CHANNELS)[:N]


# speedup vs baseline: 33.9008x; 3.2849x over previous
import functools

import jax
import jax.numpy as jnp
from jax import lax
from jax.experimental import pallas as pl
from jax.experimental.pallas import tpu as pltpu
from jax.experimental.pallas import tpu_sc as plsc

N = 500000
CHANNELS = 64
NUM_EMB = 8
LANES = 16

IR = 128
JT, JR = 8, 8
IT = (N + IR - 1) // IR
NPAD = IT * IR

TPB = 4
BLOCKS = (IT + TPB - 1) // TPB
GRPB = TPB * IR // LANES
CELEM = TPB * IR * 4
STRIPE = TPB * IR * JR

_info = plsc.get_sparse_core_info()
NC, NS = _info.num_cores, _info.num_subcores
NW = NC * NS

_GDIMS = lax.GatherDimensionNumbers(
    offset_dims=(), collapsed_slice_dims=(0,), start_index_map=(0,)
)


def _body(coords_hbm, table_hbm, out_hbm, coords_v, table_v, tt_v, rows_v, sem_c, sem_o):
    wid = lax.axis_index("s") * NC + lax.axis_index("c")
    nb = (BLOCKS - 1 - wid) // NW + 1
    iota = jnp.arange(LANES, dtype=jnp.int32)

    col_idx = (iota & 7) * CHANNELS

    def tt_body(j, carry):
        tt_v[pl.ds(j * LANES, LANES)] = plsc.load_gather(table_v, [col_idx + j])
        return carry

    def tile_start_of(b):
        return jnp.minimum(b * TPB, IT - TPB)

    def coords_issue(b, buf):
        pltpu.async_copy(
            coords_hbm.at[pl.ds(tile_start_of(b) * IR * 4, CELEM)],
            coords_v.at[pl.ds(buf * CELEM, CELEM)],
            sem_c,
        )

    def out_wait_one():
        pltpu.make_async_copy(
            rows_v.at[pl.ds(0, STRIPE)],
            out_hbm.at[pl.ds(0, STRIPE)],
            sem_o,
        ).wait()

    coords_issue(wid, 0)
    pltpu.sync_copy(table_hbm, table_v)
    lax.fori_loop(0, CHANNELS, tt_body, 0)

    def blk_body(t, carry):
        b = wid + t * NW
        buf = t % 2
        ts = tile_start_of(b)

        @pl.when(t >= 2)
        def _drain():
            for _ in range(JT):
                out_wait_one()

        pltpu.make_async_copy(
            coords_hbm.at[pl.ds(ts * IR * 4, CELEM)],
            coords_v.at[pl.ds(buf * CELEM, CELEM)],
            sem_c,
        ).wait()

        @pl.when(t + 1 < nb)
        def _prefetch():
            coords_issue(b + NW, 1 - buf)

        cbase = buf * CELEM
        rbase = buf * (JT * STRIPE)

        @plsc.parallel_loop(0, GRPB, 1, unroll=2)
        def grp_body(g):
            cb = cbase + (g // 8) * (4 * IR) + (g % 8) * LANES
            c1 = coords_v[pl.ds(cb + 1 * IR, LANES)]
            c2 = coords_v[pl.ds(cb + 2 * IR, LANES)]
            c3 = coords_v[pl.ds(cb + 3 * IR, LANES)]
            idx = (c1 & 1) | ((c2 & 1) << 1) | ((c3 & 1) << 2)
            sidx = idx[:, None]
            goff = rbase + (g // 8) * (JR * IR) + (g % 8) * LANES
            for j in range(CHANNELS):
                tcol = tt_v[pl.ds(j * LANES, LANES)]
                v = lax.gather(
                    tcol, sidx, _GDIMS, (1,),
                    mode=lax.GatherScatterMode.PROMISE_IN_BOUNDS,
                )
                rows_v[pl.ds(goff + (j // JR) * STRIPE + (j % JR) * IR, LANES)] = v

        for jt in range(JT):
            pltpu.async_copy(
                rows_v.at[pl.ds(rbase + jt * STRIPE, STRIPE)],
                out_hbm.at[pl.ds(jt * (IT * JR * IR) + ts * (JR * IR), STRIPE)],
                sem_o,
            )
        return carry

    lax.fori_loop(0, nb, blk_body, 0)

    @pl.when(nb >= 1)
    def _drain_last():
        for _ in range(JT):
            out_wait_one()

    @pl.when(nb >= 2)
    def _drain_prev():
        for _ in range(JT):
            out_wait_one()


@functools.partial(jax.jit, donate_argnums=())
def kernel(x_coords, emb_table):
    mesh = plsc.VectorSubcoreMesh(core_axis_name="c", subcore_axis_name="s")
    f = functools.partial(
        pl.kernel,
        out_type=jax.ShapeDtypeStruct((JT * IT * JR * IR,), jnp.float32),
        mesh=mesh,
        compiler_params=pltpu.CompilerParams(
            needs_layout_passes=False, use_tc_tiling_on_sc=False
        ),
        scratch_types=[
            pltpu.VMEM((2 * CELEM,), jnp.int32),
            pltpu.VMEM((NUM_EMB * CHANNELS,), jnp.float32),
            pltpu.VMEM((CHANNELS * LANES,), jnp.float32),
            pltpu.VMEM((2 * JT * STRIPE,), jnp.float32),
            pltpu.SemaphoreType.DMA,
            pltpu.SemaphoreType.DMA,
        ],
    )(_body)
    coords_pad = jnp.pad(x_coords, ((0, NPAD - N), (0, 0)))
    coords_flat = coords_pad.reshape(IT, IR, 4).transpose(0, 2, 1).reshape(-1)
    out_flat = f(coords_flat, emb_table.reshape(-1))
    p = out_flat.reshape(JT, IT, JR, IR)
    return p.transpose(1, 3, 0, 2).reshape(NPAD, CHANNELS)[:N]


# speedup vs baseline: 46.0301x; 1.3578x over previous
import functools

import jax
import jax.numpy as jnp
from jax import lax
from jax.experimental import pallas as pl
from jax.experimental.pallas import tpu as pltpu
from jax.experimental.pallas import tpu_sc as plsc

N = 500000
CHANNELS = 64
NUM_EMB = 8
LANES = 16

IR = 128
JT, JR = 8, 8
IT = (N + IR - 1) // IR
NPAD = IT * IR

TPB = 4
BLOCKS = (IT + TPB - 1) // TPB
GRPB = TPB * IR // LANES
CELEM = TPB * IR * 4
TILE_C = IR * 4
STRIPE = TPB * IR * JR

_info = plsc.get_sparse_core_info()
NC, NS = _info.num_cores, _info.num_subcores
NW = NC * NS

_GDIMS = lax.GatherDimensionNumbers(
    offset_dims=(), collapsed_slice_dims=(0,), start_index_map=(0,)
)


def _body(coords_hbm, ctail_hbm, table_hbm, out_hbm, coords_v, table_v, tt_v, rows_v, sem_c, sem_o):
    wid = lax.axis_index("s") * NC + lax.axis_index("c")
    nb = (BLOCKS - 1 - wid) // NW + 1
    iota = jnp.arange(LANES, dtype=jnp.int32)

    col_idx = (iota & 7) * CHANNELS

    def tt_body(j, carry):
        tt_v[pl.ds(j * LANES, LANES)] = plsc.load_gather(table_v, [col_idx + j])
        return carry

    def tile_start_of(b):
        return jnp.minimum(b * TPB, IT - TPB)

    def coords_issue(b, buf):
        ts = tile_start_of(b)

        @pl.when(b < BLOCKS - 1)
        def _full():
            pltpu.async_copy(
                coords_hbm.at[pl.ds(ts, TPB)], coords_v.at[buf], sem_c
            )

        @pl.when(b == BLOCKS - 1)
        def _last():
            pltpu.async_copy(
                coords_hbm.at[pl.ds(ts, TPB - 1)],
                coords_v.at[buf, pl.ds(0, TPB - 1)],
                sem_c,
            )
            pltpu.async_copy(ctail_hbm, coords_v.at[buf, TPB - 1], sem_c)

    def coords_wait(b, buf):
        @pl.when(b < BLOCKS - 1)
        def _full():
            pltpu.make_async_copy(
                coords_hbm.at[pl.ds(0, TPB)], coords_v.at[buf], sem_c
            ).wait()

        @pl.when(b == BLOCKS - 1)
        def _last():
            pltpu.make_async_copy(
                coords_hbm.at[pl.ds(0, TPB - 1)],
                coords_v.at[buf, pl.ds(0, TPB - 1)],
                sem_c,
            ).wait()
            pltpu.make_async_copy(
                ctail_hbm, coords_v.at[buf, TPB - 1], sem_c
            ).wait()

    def out_wait_one():
        pltpu.make_async_copy(
            rows_v.at[pl.ds(0, STRIPE)],
            out_hbm.at[pl.ds(0, STRIPE)],
            sem_o,
        ).wait()

    coords_issue(wid, 0)
    pltpu.sync_copy(table_hbm, table_v)
    lax.fori_loop(0, CHANNELS, tt_body, 0)

    def blk_body(t, carry):
        b = wid + t * NW
        buf = t % 2
        ts = tile_start_of(b)

        @pl.when(t >= 2)
        def _drain():
            for _ in range(JT):
                out_wait_one()

        coords_wait(b, buf)

        @pl.when(t + 1 < nb)
        def _prefetch():
            coords_issue(b + NW, 1 - buf)

        rbase = buf * (JT * STRIPE)

        @plsc.parallel_loop(0, GRPB, 1, unroll=2)
        def grp_body(g):
            tl, sl = g // 8, (g % 8) * LANES
            c1 = coords_v[buf, tl, 1, pl.ds(sl, LANES)]
            c2 = coords_v[buf, tl, 2, pl.ds(sl, LANES)]
            c3 = coords_v[buf, tl, 3, pl.ds(sl, LANES)]
            idx = (c1 & 1) | ((c2 & 1) << 1) | ((c3 & 1) << 2)
            sidx = idx[:, None]
            goff = rbase + (g // 8) * (JR * IR) + (g % 8) * LANES
            for j in range(CHANNELS):
                tcol = tt_v[pl.ds(j * LANES, LANES)]
                v = lax.gather(
                    tcol, sidx, _GDIMS, (1,),
                    mode=lax.GatherScatterMode.PROMISE_IN_BOUNDS,
                )
                rows_v[pl.ds(goff + (j // JR) * STRIPE + (j % JR) * IR, LANES)] = v

        for jt in range(JT):
            pltpu.async_copy(
                rows_v.at[pl.ds(rbase + jt * STRIPE, STRIPE)],
                out_hbm.at[pl.ds(jt * (IT * JR * IR) + ts * (JR * IR), STRIPE)],
                sem_o,
            )
        return carry

    lax.fori_loop(0, nb, blk_body, 0)

    @pl.when(nb >= 1)
    def _drain_last():
        for _ in range(JT):
            out_wait_one()

    @pl.when(nb >= 2)
    def _drain_prev():
        for _ in range(JT):
            out_wait_one()


@functools.partial(jax.jit, donate_argnums=())
def kernel(x_coords, emb_table):
    mesh = plsc.VectorSubcoreMesh(core_axis_name="c", subcore_axis_name="s")
    f = functools.partial(
        pl.kernel,
        out_type=jax.ShapeDtypeStruct((JT * IT * JR * IR,), jnp.float32),
        mesh=mesh,
        compiler_params=pltpu.CompilerParams(
            needs_layout_passes=False, use_tc_tiling_on_sc=False
        ),
        scratch_types=[
            pltpu.VMEM((2, TPB, 4, IR), jnp.int32),
            pltpu.VMEM((NUM_EMB * CHANNELS,), jnp.float32),
            pltpu.VMEM((CHANNELS * LANES,), jnp.float32),
            pltpu.VMEM((2 * JT * STRIPE,), jnp.float32),
            pltpu.SemaphoreType.DMA,
            pltpu.SemaphoreType.DMA,
        ],
    )(_body)
    nfull = (IT - 1) * IR
    coords_main = x_coords[:nfull].reshape(IT - 1, IR, 4).transpose(0, 2, 1)
    coords_tail = jnp.pad(x_coords[nfull:], ((0, IR - (N - nfull)), (0, 0))).transpose(1, 0)
    out_flat = f(coords_main, coords_tail, emb_table.reshape(-1))
    p = out_flat.reshape(JT, IT, JR, IR)
    return p.transpose(1, 3, 0, 2).reshape(NPAD, CHANNELS)[:N]


# speedup vs baseline: 46.1382x; 1.0023x over previous
import functools

import jax
import jax.numpy as jnp
from jax import lax
from jax.experimental import pallas as pl
from jax.experimental.pallas import tpu as pltpu
from jax.experimental.pallas import tpu_sc as plsc

N = 500000
CHANNELS = 64
NUM_EMB = 8
LANES = 16

IR = 128
JT, JR = 8, 8
IT = (N + IR - 1) // IR
NPAD = IT * IR

TPB = 4
BLOCKS = (IT + TPB - 1) // TPB
GRPB = TPB * IR // LANES
CELEM = TPB * IR * 4
TILE_C = IR * 4
STRIPE = TPB * IR * JR

_info = plsc.get_sparse_core_info()
NC, NS = _info.num_cores, _info.num_subcores
NW = NC * NS

_GDIMS = lax.GatherDimensionNumbers(
    offset_dims=(), collapsed_slice_dims=(0,), start_index_map=(0,)
)


def _body(coords_hbm, ctail_hbm, table_hbm, out_hbm, coords_v, table_v, tt_v, rows_v, sem_c, sem_o):
    wid = lax.axis_index("s") * NC + lax.axis_index("c")
    nb = (BLOCKS - 1 - wid) // NW + 1
    iota = jnp.arange(LANES, dtype=jnp.int32)

    col_idx = (iota & 7) * CHANNELS

    def tt_body(j, carry):
        tt_v[pl.ds(j * LANES, LANES)] = plsc.load_gather(table_v, [col_idx + j])
        return carry

    def tile_start_of(b):
        return jnp.minimum(b * TPB, IT - TPB)

    def coords_issue(b, buf):
        ts = tile_start_of(b)

        @pl.when(b < BLOCKS - 1)
        def _full():
            pltpu.async_copy(
                coords_hbm.at[pl.ds(ts, TPB)], coords_v.at[buf], sem_c
            )

        @pl.when(b == BLOCKS - 1)
        def _last():
            pltpu.async_copy(
                coords_hbm.at[pl.ds(ts, TPB - 1)],
                coords_v.at[buf, pl.ds(0, TPB - 1)],
                sem_c,
            )
            pltpu.async_copy(ctail_hbm, coords_v.at[buf, TPB - 1], sem_c)

    def coords_wait(b, buf):
        @pl.when(b < BLOCKS - 1)
        def _full():
            pltpu.make_async_copy(
                coords_hbm.at[pl.ds(0, TPB)], coords_v.at[buf], sem_c
            ).wait()

        @pl.when(b == BLOCKS - 1)
        def _last():
            pltpu.make_async_copy(
                coords_hbm.at[pl.ds(0, TPB - 1)],
                coords_v.at[buf, pl.ds(0, TPB - 1)],
                sem_c,
            ).wait()
            pltpu.make_async_copy(
                ctail_hbm, coords_v.at[buf, TPB - 1], sem_c
            ).wait()

    def out_wait_one():
        pltpu.make_async_copy(
            rows_v.at[pl.ds(0, JT * STRIPE)],
            out_hbm.at[pl.ds(0, JT * STRIPE)],
            sem_o,
        ).wait()

    coords_issue(wid, 0)
    pltpu.sync_copy(table_hbm, table_v)
    lax.fori_loop(0, CHANNELS, tt_body, 0)

    def blk_body(t, carry):
        b = wid + t * NW
        buf = t % 2
        ts = tile_start_of(b)

        @pl.when(t >= 2)
        def _drain():
            out_wait_one()

        coords_wait(b, buf)

        @pl.when(t + 1 < nb)
        def _prefetch():
            coords_issue(b + NW, 1 - buf)

        rbase = buf * (JT * STRIPE)

        @plsc.parallel_loop(0, GRPB, 1, unroll=2)
        def grp_body(g):
            tl, sl = g // 8, (g % 8) * LANES
            c1 = coords_v[buf, tl, 1, pl.ds(sl, LANES)]
            c2 = coords_v[buf, tl, 2, pl.ds(sl, LANES)]
            c3 = coords_v[buf, tl, 3, pl.ds(sl, LANES)]
            idx = (c1 & 1) | ((c2 & 1) << 1) | ((c3 & 1) << 2)
            sidx = idx[:, None]
            goff = rbase + (g // 8) * (JR * IR) + (g % 8) * LANES
            for j in range(CHANNELS):
                tcol = tt_v[pl.ds(j * LANES, LANES)]
                v = lax.gather(
                    tcol, sidx, _GDIMS, (1,),
                    mode=lax.GatherScatterMode.PROMISE_IN_BOUNDS,
                )
                rows_v[pl.ds(goff + (j // JR) * STRIPE + (j % JR) * IR, LANES)] = v

        for jt in range(JT):
            pltpu.async_copy(
                rows_v.at[pl.ds(rbase + jt * STRIPE, STRIPE)],
                out_hbm.at[pl.ds(jt * (IT * JR * IR) + ts * (JR * IR), STRIPE)],
                sem_o,
            )
        return carry

    lax.fori_loop(0, nb, blk_body, 0)

    @pl.when(nb >= 1)
    def _drain_last():
        out_wait_one()

    @pl.when(nb >= 2)
    def _drain_prev():
        out_wait_one()


@functools.partial(jax.jit, donate_argnums=())
def kernel(x_coords, emb_table):
    mesh = plsc.VectorSubcoreMesh(core_axis_name="c", subcore_axis_name="s")
    f = functools.partial(
        pl.kernel,
        out_type=jax.ShapeDtypeStruct((JT * IT * JR * IR,), jnp.float32),
        mesh=mesh,
        compiler_params=pltpu.CompilerParams(
            needs_layout_passes=False, use_tc_tiling_on_sc=False
        ),
        scratch_types=[
            pltpu.VMEM((2, TPB, 4, IR), jnp.int32),
            pltpu.VMEM((NUM_EMB * CHANNELS,), jnp.float32),
            pltpu.VMEM((CHANNELS * LANES,), jnp.float32),
            pltpu.VMEM((2 * JT * STRIPE,), jnp.float32),
            pltpu.SemaphoreType.DMA,
            pltpu.SemaphoreType.DMA,
        ],
    )(_body)
    nfull = (IT - 1) * IR
    coords_main = x_coords[:nfull].reshape(IT - 1, IR, 4).transpose(0, 2, 1)
    coords_tail = jnp.pad(x_coords[nfull:], ((0, IR - (N - nfull)), (0, 0))).transpose(1, 0)
    out_flat = f(coords_main, coords_tail, emb_table.reshape(-1))
    p = out_flat.reshape(JT, IT, JR, IR)
    return p.transpose(1, 3, 0, 2).reshape(NPAD, CHANNELS)[:N]
